# baseline (speedup 1.0000x reference)
import functools

import jax
import jax.numpy as jnp
from jax import lax
from jax.experimental import pallas as pl
from jax.experimental.pallas import tpu as pltpu
from jax.experimental.pallas import tpu_sc as plsc

DIM = 4096
N_EXP = 64
TOPK = 8
N_GROUPS = 8
GROUP_SIZE = N_EXP // N_GROUPS
TOPK_GROUPS = 4
ROUTE_SCALE = 2.5
N_TOK = 32768

BLOCK_T = 1024

NC = 2
NS = 16
NW = NC * NS
C_PER_W = N_TOK // NW
LANES = 16


def _scores_body(x_ref, w_ref, s_ref):
    logits_t = jax.lax.dot_general(
        w_ref[...], x_ref[...], (((1,), (1,)), ((), ())),
        preferred_element_type=jnp.float32,
    )
    s_ref[...] = jax.nn.sigmoid(logits_t)


def _tc_scores(x, W, tok0, cn):
    blk0 = tok0 // BLOCK_T
    return pl.pallas_call(
        _scores_body,
        grid=(cn // BLOCK_T,),
        in_specs=[
            pl.BlockSpec((BLOCK_T, DIM), lambda i: (blk0 + i, 0)),
            pl.BlockSpec((N_EXP, DIM), lambda i: (0, 0)),
        ],
        out_specs=pl.BlockSpec((N_EXP, BLOCK_T), lambda i: (0, i)),
        out_shape=jax.ShapeDtypeStruct((N_EXP, cn), jnp.float32),
    )(x, W)


def _route_body(c_per_w, s_hbm, wout_hbm, iout_hbm, sv, wv, iv):
    wid = lax.axis_index("s") * NC + lax.axis_index("c")
    base = wid * c_per_w
    pltpu.sync_copy(s_hbm.at[:, pl.ds(base, c_per_w)], sv)

    def chunk(c, carry):
        o = c * LANES
        tok = o + lax.iota(jnp.int32, LANES)

        gm = []
        for g in range(N_GROUPS):
            m = sv[g * GROUP_SIZE, pl.ds(o, LANES)]
            for j in range(1, GROUP_SIZE):
                m = jnp.maximum(m, sv[g * GROUP_SIZE + j, pl.ds(o, LANES)])
            gm.append(m)

        gsel = []
        for _ in range(TOPK_GROUPS):
            m = gm[0]
            for g in range(1, N_GROUPS):
                m = jnp.maximum(m, gm[g])
            gidx = jnp.full((LANES,), N_GROUPS, jnp.int32)
            for g in range(N_GROUPS - 1, -1, -1):
                gidx = jnp.where(gm[g] == m, g, gidx)
            gsel.append(gidx)
            for g in range(N_GROUPS):
                gm[g] = jnp.where(gidx == g, -1.0, gm[g])

        for a, b in ((0, 1), (2, 3), (0, 2), (1, 3), (1, 2)):
            lo = jnp.minimum(gsel[a], gsel[b])
            hi = jnp.maximum(gsel[a], gsel[b])
            gsel[a], gsel[b] = lo, hi

        slot_v = [jnp.full((LANES,), -1.0, jnp.float32) for _ in range(TOPK)]
        slot_i = [jnp.full((LANES,), N_EXP, jnp.int32) for _ in range(TOPK)]
        for r in range(TOPK_GROUPS):
            for j in range(GROUP_SIZE):
                ci = gsel[r] * GROUP_SIZE + j
                cv = plsc.load_gather(sv, [ci, tok])
                beats = [cv > slot_v[k] for k in range(TOPK)]
                for k in range(TOPK - 1, 0, -1):
                    ins_v = jnp.where(beats[k], cv, slot_v[k])
                    ins_i = jnp.where(beats[k], ci, slot_i[k])
                    slot_v[k] = jnp.where(beats[k - 1], slot_v[k - 1], ins_v)
                    slot_i[k] = jnp.where(beats[k - 1], slot_i[k - 1], ins_i)
                slot_v[0] = jnp.where(beats[0], cv, slot_v[0])
                slot_i[0] = jnp.where(beats[0], ci, slot_i[0])

        tot = ((slot_v[0] + slot_v[1]) + (slot_v[2] + slot_v[3])) + (
            (slot_v[4] + slot_v[5]) + (slot_v[6] + slot_v[7]))
        for k in range(TOPK):
            wk = (slot_v[k] / tot) * ROUTE_SCALE
            kvec = jnp.full((LANES,), k, jnp.int32)
            plsc.store_scatter(wv, [tok, kvec], wk)
            plsc.store_scatter(iv, [tok, kvec], slot_i[k])
        return carry

    lax.fori_loop(0, c_per_w // LANES, chunk, 0)
    pltpu.sync_copy(wv, wout_hbm.at[pl.ds(base, c_per_w)])
    pltpu.sync_copy(iv, iout_hbm.at[pl.ds(base, c_per_w)])


def _sc_route(scores_t):
    n_tok = scores_t.shape[1]
    c_per_w = n_tok // NW
    mesh = plsc.VectorSubcoreMesh(core_axis_name="c", subcore_axis_name="s")
    f = pl.kernel(
        functools.partial(_route_body, c_per_w),
        out_type=[
            jax.ShapeDtypeStruct((n_tok, TOPK), jnp.float32),
            jax.ShapeDtypeStruct((n_tok, TOPK), jnp.int32),
        ],
        mesh=mesh,
        compiler_params=pltpu.CompilerParams(
            use_tc_tiling_on_sc=False, needs_layout_passes=False),
        scratch_types=[
            pltpu.VMEM((N_EXP, c_per_w), jnp.float32),
            pltpu.VMEM((c_per_w, TOPK), jnp.float32),
            pltpu.VMEM((c_per_w, TOPK), jnp.int32),
        ],
    )
    return f(scores_t)


CHUNK_SIZES = (26624, 6144)


def kernel(x, W):
    n = len(CHUNK_SIZES)
    scores = [None] * n
    w_parts, i_parts = [None] * n, [None] * n
    starts = [sum(CHUNK_SIZES[:c]) for c in range(n)]
    scores[0] = _tc_scores(x, W, starts[0], CHUNK_SIZES[0])
    for c in range(n):
        if c + 1 < n:
            scores[c + 1] = _tc_scores(x, W, starts[c + 1], CHUNK_SIZES[c + 1])
        w_parts[c], i_parts[c] = _sc_route(scores[c])
    return jnp.concatenate(w_parts, axis=0), jnp.concatenate(i_parts, axis=0)


# speedup vs baseline: 1.0590x; 1.0590x over previous
import functools

import jax
import jax.numpy as jnp
from jax import lax
from jax.experimental import pallas as pl
from jax.experimental.pallas import tpu as pltpu
from jax.experimental.pallas import tpu_sc as plsc

DIM = 4096
N_EXP = 64
TOPK = 8
N_GROUPS = 8
GROUP_SIZE = N_EXP // N_GROUPS
TOPK_GROUPS = 4
ROUTE_SCALE = 2.5
N_TOK = 32768

BLOCK_T = 1024

NC = 2
NS = 16
NW = NC * NS
C_PER_W = N_TOK // NW
LANES = 16


def _scores_body(x_ref, w_ref, s_ref):
    logits_t = jax.lax.dot_general(
        w_ref[...], x_ref[...], (((1,), (1,)), ((), ())),
        preferred_element_type=jnp.float32,
    )
    s_ref[...] = jax.nn.sigmoid(logits_t)


def _tc_scores(x, W, tok0, cn):
    blk0 = tok0 // BLOCK_T
    return pl.pallas_call(
        _scores_body,
        grid=(cn // BLOCK_T,),
        in_specs=[
            pl.BlockSpec((BLOCK_T, DIM), lambda i: (blk0 + i, 0)),
            pl.BlockSpec((N_EXP, DIM), lambda i: (0, 0)),
        ],
        out_specs=pl.BlockSpec((N_EXP, BLOCK_T), lambda i: (0, i)),
        out_shape=jax.ShapeDtypeStruct((N_EXP, cn), jnp.float32),
    )(x, W)


def _route_body(c_per_w, s_hbm, wout_hbm, iout_hbm, sv, wv, iv):
    wid = lax.axis_index("s") * NC + lax.axis_index("c")
    base = wid * c_per_w
    pltpu.sync_copy(s_hbm.at[:, pl.ds(base, c_per_w)], sv)

    def chunk(c, carry):
        o = c * LANES
        tok = o + lax.iota(jnp.int32, LANES)

        gm = []
        for g in range(N_GROUPS):
            m = sv[g * GROUP_SIZE, pl.ds(o, LANES)]
            for j in range(1, GROUP_SIZE):
                m = jnp.maximum(m, sv[g * GROUP_SIZE + j, pl.ds(o, LANES)])
            gm.append(m)

        gsel = []
        for _ in range(TOPK_GROUPS):
            m = gm[0]
            for g in range(1, N_GROUPS):
                m = jnp.maximum(m, gm[g])
            gidx = jnp.full((LANES,), N_GROUPS, jnp.int32)
            for g in range(N_GROUPS - 1, -1, -1):
                gidx = jnp.where(gm[g] == m, g, gidx)
            gsel.append(gidx)
            for g in range(N_GROUPS):
                gm[g] = jnp.where(gidx == g, -1.0, gm[g])

        for a, b in ((0, 1), (2, 3), (0, 2), (1, 3), (1, 2)):
            lo = jnp.minimum(gsel[a], gsel[b])
            hi = jnp.maximum(gsel[a], gsel[b])
            gsel[a], gsel[b] = lo, hi

        slot_v = [jnp.full((LANES,), -1.0, jnp.float32) for _ in range(TOPK)]
        slot_i = [jnp.full((LANES,), N_EXP, jnp.int32) for _ in range(TOPK)]
        for r in range(TOPK_GROUPS):
            for j in range(GROUP_SIZE):
                ci = gsel[r] * GROUP_SIZE + j
                cv = plsc.load_gather(sv, [ci, tok])
                beats = [cv > slot_v[k] for k in range(TOPK)]
                for k in range(TOPK - 1, 0, -1):
                    ins_v = jnp.where(beats[k], cv, slot_v[k])
                    ins_i = jnp.where(beats[k], ci, slot_i[k])
                    slot_v[k] = jnp.where(beats[k - 1], slot_v[k - 1], ins_v)
                    slot_i[k] = jnp.where(beats[k - 1], slot_i[k - 1], ins_i)
                slot_v[0] = jnp.where(beats[0], cv, slot_v[0])
                slot_i[0] = jnp.where(beats[0], ci, slot_i[0])

        tot = ((slot_v[0] + slot_v[1]) + (slot_v[2] + slot_v[3])) + (
            (slot_v[4] + slot_v[5]) + (slot_v[6] + slot_v[7]))
        for k in range(TOPK):
            wk = (slot_v[k] / tot) * ROUTE_SCALE
            kvec = jnp.full((LANES,), k, jnp.int32)
            plsc.store_scatter(wv, [tok, kvec], wk)
            plsc.store_scatter(iv, [tok, kvec], slot_i[k])
        return carry

    lax.fori_loop(0, c_per_w // LANES, chunk, 0)
    pltpu.sync_copy(wv, wout_hbm.at[pl.ds(base, c_per_w)])
    pltpu.sync_copy(iv, iout_hbm.at[pl.ds(base, c_per_w)])


def _sc_route(scores_t):
    n_tok = scores_t.shape[1]
    c_per_w = n_tok // NW
    mesh = plsc.VectorSubcoreMesh(core_axis_name="c", subcore_axis_name="s")
    f = pl.kernel(
        functools.partial(_route_body, c_per_w),
        out_type=[
            jax.ShapeDtypeStruct((n_tok, TOPK), jnp.float32),
            jax.ShapeDtypeStruct((n_tok, TOPK), jnp.int32),
        ],
        mesh=mesh,
        compiler_params=pltpu.CompilerParams(
            use_tc_tiling_on_sc=False, needs_layout_passes=False),
        scratch_types=[
            pltpu.VMEM((N_EXP, c_per_w), jnp.float32),
            pltpu.VMEM((c_per_w, TOPK), jnp.float32),
            pltpu.VMEM((c_per_w, TOPK), jnp.int32),
        ],
    )
    return f(scores_t)


CHUNK_SIZES = (16384, 16384)


def kernel(x, W):
    n = len(CHUNK_SIZES)
    scores = [None] * n
    w_parts, i_parts = [None] * n, [None] * n
    starts = [sum(CHUNK_SIZES[:c]) for c in range(n)]
    scores[0] = _tc_scores(x, W, starts[0], CHUNK_SIZES[0])
    for c in range(n):
        if c + 1 < n:
            scores[c + 1] = _tc_scores(x, W, starts[c + 1], CHUNK_SIZES[c + 1])
        w_parts[c], i_parts[c] = _sc_route(scores[c])
    return jnp.concatenate(w_parts, axis=0), jnp.concatenate(i_parts, axis=0)


# speedup vs baseline: 1.0597x; 1.0007x over previous
import functools

import jax
import jax.numpy as jnp
from jax import lax
from jax.experimental import pallas as pl
from jax.experimental.pallas import tpu as pltpu
from jax.experimental.pallas import tpu_sc as plsc

DIM = 4096
N_EXP = 64
TOPK = 8
N_GROUPS = 8
GROUP_SIZE = N_EXP // N_GROUPS
TOPK_GROUPS = 4
ROUTE_SCALE = 2.5
N_TOK = 32768

BLOCK_T = 1024

NC = 2
NS = 16
NW = NC * NS
C_PER_W = N_TOK // NW
LANES = 16


def _scores_body(x_ref, w_ref, s_ref):
    logits_t = jax.lax.dot_general(
        w_ref[...], x_ref[...], (((1,), (1,)), ((), ())),
        preferred_element_type=jnp.float32,
    )
    s_ref[...] = jax.nn.sigmoid(logits_t)


def _tc_scores(x, W, tok0, cn):
    blk0 = tok0 // BLOCK_T
    return pl.pallas_call(
        _scores_body,
        grid=(cn // BLOCK_T,),
        in_specs=[
            pl.BlockSpec((BLOCK_T, DIM), lambda i: (blk0 + i, 0)),
            pl.BlockSpec((N_EXP, DIM), lambda i: (0, 0)),
        ],
        out_specs=pl.BlockSpec((N_EXP, BLOCK_T), lambda i: (0, i)),
        out_shape=jax.ShapeDtypeStruct((N_EXP, cn), jnp.float32),
    )(x, W)


def _route_body(c_per_w, s_hbm, wout_hbm, iout_hbm, sv, wv, iv):
    wid = lax.axis_index("s") * NC + lax.axis_index("c")
    base = wid * c_per_w
    pltpu.sync_copy(s_hbm.at[:, pl.ds(base, c_per_w)], sv)

    def route16(o):
        tok = o + lax.iota(jnp.int32, LANES)

        gm = []
        for g in range(N_GROUPS):
            m = sv[g * GROUP_SIZE, pl.ds(o, LANES)]
            for j in range(1, GROUP_SIZE):
                m = jnp.maximum(m, sv[g * GROUP_SIZE + j, pl.ds(o, LANES)])
            gm.append(m)

        gsel = []
        for _ in range(TOPK_GROUPS):
            m = gm[0]
            for g in range(1, N_GROUPS):
                m = jnp.maximum(m, gm[g])
            gidx = jnp.full((LANES,), N_GROUPS, jnp.int32)
            for g in range(N_GROUPS - 1, -1, -1):
                gidx = jnp.where(gm[g] == m, g, gidx)
            gsel.append(gidx)
            for g in range(N_GROUPS):
                gm[g] = jnp.where(gidx == g, -1.0, gm[g])

        for a, b in ((0, 1), (2, 3), (0, 2), (1, 3), (1, 2)):
            lo = jnp.minimum(gsel[a], gsel[b])
            hi = jnp.maximum(gsel[a], gsel[b])
            gsel[a], gsel[b] = lo, hi

        slot_v = [jnp.full((LANES,), -1.0, jnp.float32) for _ in range(TOPK)]
        slot_i = [jnp.full((LANES,), N_EXP, jnp.int32) for _ in range(TOPK)]
        for r in range(TOPK_GROUPS):
            for j in range(GROUP_SIZE):
                ci = gsel[r] * GROUP_SIZE + j
                cv = plsc.load_gather(sv, [ci, tok])
                beats = [cv > slot_v[k] for k in range(TOPK)]
                for k in range(TOPK - 1, 0, -1):
                    ins_v = jnp.where(beats[k], cv, slot_v[k])
                    ins_i = jnp.where(beats[k], ci, slot_i[k])
                    slot_v[k] = jnp.where(beats[k - 1], slot_v[k - 1], ins_v)
                    slot_i[k] = jnp.where(beats[k - 1], slot_i[k - 1], ins_i)
                slot_v[0] = jnp.where(beats[0], cv, slot_v[0])
                slot_i[0] = jnp.where(beats[0], ci, slot_i[0])

        tot = ((slot_v[0] + slot_v[1]) + (slot_v[2] + slot_v[3])) + (
            (slot_v[4] + slot_v[5]) + (slot_v[6] + slot_v[7]))
        for k in range(TOPK):
            wk = (slot_v[k] / tot) * ROUTE_SCALE
            kvec = jnp.full((LANES,), k, jnp.int32)
            plsc.store_scatter(wv, [tok, kvec], wk)
            plsc.store_scatter(iv, [tok, kvec], slot_i[k])

    def chunk(c, carry):
        route16(c * (2 * LANES))
        route16(c * (2 * LANES) + LANES)
        return carry

    lax.fori_loop(0, c_per_w // (2 * LANES), chunk, 0)
    pltpu.sync_copy(wv, wout_hbm.at[pl.ds(base, c_per_w)])
    pltpu.sync_copy(iv, iout_hbm.at[pl.ds(base, c_per_w)])


def _sc_route(scores_t):
    n_tok = scores_t.shape[1]
    c_per_w = n_tok // NW
    mesh = plsc.VectorSubcoreMesh(core_axis_name="c", subcore_axis_name="s")
    f = pl.kernel(
        functools.partial(_route_body, c_per_w),
        out_type=[
            jax.ShapeDtypeStruct((n_tok, TOPK), jnp.float32),
            jax.ShapeDtypeStruct((n_tok, TOPK), jnp.int32),
        ],
        mesh=mesh,
        compiler_params=pltpu.CompilerParams(
            use_tc_tiling_on_sc=False, needs_layout_passes=False),
        scratch_types=[
            pltpu.VMEM((N_EXP, c_per_w), jnp.float32),
            pltpu.VMEM((c_per_w, TOPK), jnp.float32),
            pltpu.VMEM((c_per_w, TOPK), jnp.int32),
        ],
    )
    return f(scores_t)


CHUNK_SIZES = (16384, 16384)


def kernel(x, W):
    n = len(CHUNK_SIZES)
    scores = [None] * n
    w_parts, i_parts = [None] * n, [None] * n
    starts = [sum(CHUNK_SIZES[:c]) for c in range(n)]
    scores[0] = _tc_scores(x, W, starts[0], CHUNK_SIZES[0])
    for c in range(n):
        if c + 1 < n:
            scores[c + 1] = _tc_scores(x, W, starts[c + 1], CHUNK_SIZES[c + 1])
        w_parts[c], i_parts[c] = _sc_route(scores[c])
    return jnp.concatenate(w_parts, axis=0), jnp.concatenate(i_parts, axis=0)


# speedup vs baseline: 1.2197x; 1.1510x over previous
import functools

import jax
import jax.numpy as jnp
from jax import lax
from jax.experimental import pallas as pl
from jax.experimental.pallas import tpu as pltpu
from jax.experimental.pallas import tpu_sc as plsc

DIM = 4096
N_EXP = 64
TOPK = 8
N_GROUPS = 8
GROUP_SIZE = N_EXP // N_GROUPS
TOPK_GROUPS = 4
ROUTE_SCALE = 2.5
N_TOK = 32768

BLOCK_T = 1024

NC = 2
NS = 16
NW = NC * NS
LANES = 16


def _scores_body(x_ref, w_ref, s_ref):
    logits_t = jax.lax.dot_general(
        w_ref[...], x_ref[...], (((1,), (1,)), ((), ())),
        preferred_element_type=jnp.float32,
    )
    s_ref[...] = jax.nn.sigmoid(logits_t)


def _tc_scores(x, W, tok0, cn):
    blk0 = tok0 // BLOCK_T
    return pl.pallas_call(
        _scores_body,
        grid=(cn // BLOCK_T,),
        in_specs=[
            pl.BlockSpec((BLOCK_T, DIM), lambda i: (blk0 + i, 0)),
            pl.BlockSpec((N_EXP, DIM), lambda i: (0, 0)),
        ],
        out_specs=pl.BlockSpec((N_EXP, BLOCK_T), lambda i: (0, i)),
        out_shape=jax.ShapeDtypeStruct((N_EXP, cn), jnp.float32),
    )(x, W)


def _route_body(c_per_w, s_hbm, wout_hbm, iout_hbm, sv, wv, iv):
    wid = lax.axis_index("s") * NC + lax.axis_index("c")
    base = wid * c_per_w
    pltpu.sync_copy(s_hbm.at[:, pl.ds(base, c_per_w)], sv)

    def route16(o):
        gm = []
        for g in range(N_GROUPS):
            m = sv[g * GROUP_SIZE, pl.ds(o, LANES)]
            for j in range(1, GROUP_SIZE):
                m = jnp.maximum(m, sv[g * GROUP_SIZE + j, pl.ds(o, LANES)])
            gm.append(m)

        gsel = []
        for _ in range(TOPK_GROUPS):
            m = gm[0]
            for g in range(1, N_GROUPS):
                m = jnp.maximum(m, gm[g])
            gidx = jnp.full((LANES,), N_GROUPS, jnp.int32)
            for g in range(N_GROUPS - 1, -1, -1):
                gidx = jnp.where(gm[g] == m, g, gidx)
            gsel.append(gidx)
            for g in range(N_GROUPS):
                gm[g] = jnp.where(gidx == g, -1.0, gm[g])

        for a, b in ((0, 1), (2, 3), (0, 2), (1, 3), (1, 2)):
            lo = jnp.minimum(gsel[a], gsel[b])
            hi = jnp.maximum(gsel[a], gsel[b])
            gsel[a], gsel[b] = lo, hi

        slot_v = [jnp.full((LANES,), -1.0, jnp.float32) for _ in range(TOPK)]
        slot_i = [jnp.full((LANES,), N_EXP, jnp.int32) for _ in range(TOPK)]
        tok = o + lax.iota(jnp.int32, LANES)
        for r in range(TOPK_GROUPS):
            for j in range(GROUP_SIZE):
                ci = gsel[r] * GROUP_SIZE + j
                cv = plsc.load_gather(sv, [ci, tok])
                beats = [cv > slot_v[k] for k in range(TOPK)]
                for k in range(TOPK - 1, 0, -1):
                    ins_v = jnp.where(beats[k], cv, slot_v[k])
                    ins_i = jnp.where(beats[k], ci, slot_i[k])
                    slot_v[k] = jnp.where(beats[k - 1], slot_v[k - 1], ins_v)
                    slot_i[k] = jnp.where(beats[k - 1], slot_i[k - 1], ins_i)
                slot_v[0] = jnp.where(beats[0], cv, slot_v[0])
                slot_i[0] = jnp.where(beats[0], ci, slot_i[0])

        tot = ((slot_v[0] + slot_v[1]) + (slot_v[2] + slot_v[3])) + (
            (slot_v[4] + slot_v[5]) + (slot_v[6] + slot_v[7]))
        for k in range(TOPK):
            wk = (slot_v[k] / tot) * ROUTE_SCALE
            wv[k, pl.ds(o, LANES)] = wk
            iv[k, pl.ds(o, LANES)] = slot_i[k]

    def chunk(c, carry):
        route16(c * (2 * LANES))
        route16(c * (2 * LANES) + LANES)
        return carry

    lax.fori_loop(0, c_per_w // (2 * LANES), chunk, 0)
    pltpu.sync_copy(wv, wout_hbm.at[:, pl.ds(base, c_per_w)])
    pltpu.sync_copy(iv, iout_hbm.at[:, pl.ds(base, c_per_w)])


def _sc_route(scores_t):
    n_tok = scores_t.shape[1]
    c_per_w = n_tok // NW
    mesh = plsc.VectorSubcoreMesh(core_axis_name="c", subcore_axis_name="s")
    f = pl.kernel(
        functools.partial(_route_body, c_per_w),
        out_type=[
            jax.ShapeDtypeStruct((TOPK, n_tok), jnp.float32),
            jax.ShapeDtypeStruct((TOPK, n_tok), jnp.int32),
        ],
        mesh=mesh,
        compiler_params=pltpu.CompilerParams(
            use_tc_tiling_on_sc=False, needs_layout_passes=False),
        scratch_types=[
            pltpu.VMEM((N_EXP, c_per_w), jnp.float32),
            pltpu.VMEM((TOPK, c_per_w), jnp.float32),
            pltpu.VMEM((TOPK, c_per_w), jnp.int32),
        ],
    )
    return f(scores_t)


CHUNK_SIZES = (16384, 16384)


def kernel(x, W):
    n = len(CHUNK_SIZES)
    scores = [None] * n
    w_parts, i_parts = [None] * n, [None] * n
    starts = [sum(CHUNK_SIZES[:c]) for c in range(n)]
    scores[0] = _tc_scores(x, W, starts[0], CHUNK_SIZES[0])
    for c in range(n):
        if c + 1 < n:
            scores[c + 1] = _tc_scores(x, W, starts[c + 1], CHUNK_SIZES[c + 1])
        w_parts[c], i_parts[c] = _sc_route(scores[c])
    return (jnp.concatenate(w_parts, axis=1).T,
            jnp.concatenate(i_parts, axis=1).T)
